# baseline (speedup 1.0000x reference)
import jax
import jax.numpy as jnp
from jax import lax
from jax.experimental import pallas as pl
from jax.experimental.pallas import tpu as pltpu
from jax.experimental.pallas import tpu_sc as plsc

T = 2048
D = 1024
DFF = 2048
E = 8
C = 640
S = E * C
S_PAD = 5376
N_TRASH = S_PAD - S
NB = S_PAD // 128

NC = 2
NS = 16
HALF = S_PAD // NC
PER_TILE = HALF // NS
INIT_PER_TILE = S_PAD // NS


def _router_body(x_ref, wg_ref, dest_ref, gate_ref, xbf_ref):
    x = x_ref[...]
    logits = lax.dot_general(wg_ref[...], x, (((0,), (1,)), ((), ())),
                             preferred_element_type=jnp.float32)
    row = lax.broadcasted_iota(jnp.int32, (E, T), 0)
    m = jnp.max(logits, axis=0, keepdims=True)
    p = jnp.exp(logits - m)
    probs = p / jnp.sum(p, axis=0, keepdims=True)
    p0 = jnp.max(probs, axis=0, keepdims=True)
    i0 = jnp.min(jnp.where(probs == p0, row, E), axis=0, keepdims=True)
    mask0 = row == i0
    probs2 = jnp.where(mask0, -1.0, probs)
    p1 = jnp.max(probs2, axis=0, keepdims=True)
    i1 = jnp.min(jnp.where(probs2 == p1, row, E), axis=0, keepdims=True)
    mask1 = row == i1
    sm = p0 + p1 + 1e-9
    g0 = p0 / sm
    g1 = p1 / sm
    cnt = mask0.astype(jnp.bfloat16) + mask1.astype(jnp.bfloat16)
    ri = lax.broadcasted_iota(jnp.int32, (T, T), 0)
    cj = lax.broadcasted_iota(jnp.int32, (T, T), 1)
    triu = (ri < cj).astype(jnp.bfloat16)
    pref = jnp.dot(cnt, triu, preferred_element_type=jnp.float32)
    pos0 = jnp.sum(jnp.where(mask0, pref, 0.0), axis=0,
                   keepdims=True).astype(jnp.int32)
    pos1 = jnp.sum(jnp.where(mask1, pref, 0.0), axis=0,
                   keepdims=True).astype(jnp.int32)
    keep0 = pos0 < C
    keep1 = pos1 < C
    t2 = lax.broadcasted_iota(jnp.int32, (1, T), 1) * 2
    d0 = jnp.where(keep0, i0 * C + pos0, S + (t2 % N_TRASH))
    d1 = jnp.where(keep1, i1 * C + pos1, S + ((t2 + 1) % N_TRASH))
    gv0 = jnp.where(keep0, g0, 0.0)
    gv1 = jnp.where(keep1, g1, 0.0)
    dest_ref[...] = jnp.concatenate([d0, d1], axis=0)
    gate_ref[...] = jnp.concatenate([gv0, gv1], axis=0)
    xbf_ref[...] = x.astype(jnp.bfloat16)


def _run_router(x, Wg):
    return pl.pallas_call(
        _router_body,
        out_shape=[jax.ShapeDtypeStruct((2, T), jnp.int32),
                   jax.ShapeDtypeStruct((2, T), jnp.float32),
                   jax.ShapeDtypeStruct((T, D), jnp.bfloat16)],
    )(x, Wg)


def _dispatch_body(dest_hbm, tok_hbm, x_hbm, buf_hbm,
                   tfs_sp, idx_v, tok_v, tfs_v, rows_a, rows_b, binit_i,
                   sem_in, sem_sp, sem_ga, sem_gb, sem_wa, sem_wb):
    cid = lax.axis_index("c")
    sid = lax.axis_index("s")
    t0 = sid * 128
    c1 = pltpu.async_copy(dest_hbm.at[:, pl.ds(t0, 128)], idx_v, sem_in)
    c2 = pltpu.async_copy(tok_hbm.at[:, pl.ds(t0, 128)], tok_v, sem_in)
    zi = jnp.zeros((16,), jnp.int32)
    for k in range(INIT_PER_TILE // 16):
        binit_i[pl.ds(k * 16, 16)] = zi
    o = sid * INIT_PER_TILE
    i1 = pltpu.async_copy(binit_i, tfs_sp.at[pl.ds(o, INIT_PER_TILE)], sem_sp)
    c1.wait()
    c2.wait()
    i1.wait()
    plsc.subcore_barrier()
    s0 = pltpu.async_copy(tok_v.at[0], tfs_sp.at[idx_v.at[0]], sem_sp)
    s1 = pltpu.async_copy(tok_v.at[1], tfs_sp.at[idx_v.at[1]], sem_sp)
    s0.wait()
    s1.wait()
    plsc.subcore_barrier()
    g0 = cid * HALF + sid * PER_TILE
    pltpu.sync_copy(tfs_sp.at[pl.ds(g0, PER_TILE)], tfs_v)
    offs = (0, 88)
    szs = (88, 80)
    bufs = (rows_a, rows_b)
    gsems = (sem_ga, sem_gb)
    wsems = (sem_wa, sem_wb)

    def start_gather(c):
        return pltpu.async_copy(
            x_hbm.at[tfs_v.at[pl.ds(offs[c], szs[c])]],
            bufs[c].at[pl.ds(0, szs[c])], gsems[c])

    def start_write(c):
        return pltpu.async_copy(
            bufs[c].at[pl.ds(0, szs[c])],
            buf_hbm.at[pl.ds(g0 + offs[c], szs[c])], wsems[c])

    g_a = start_gather(0)
    g_b = start_gather(1)
    g_a.wait()
    w_a = start_write(0)
    g_b.wait()
    w_b = start_write(1)
    w_a.wait()
    w_b.wait()


def _run_dispatch(dest_t, tok_t, x_bf):
    mesh = plsc.VectorSubcoreMesh(core_axis_name="c", subcore_axis_name="s")
    f = pl.kernel(
        _dispatch_body,
        out_type=jax.ShapeDtypeStruct((S_PAD, D // 2), jnp.int32),
        mesh=mesh,
        scratch_types=[
            pltpu.VMEM_SHARED((S_PAD,), jnp.int32),
            pltpu.VMEM((2, 128), jnp.int32),
            pltpu.VMEM((2, 128), jnp.int32),
            pltpu.VMEM((PER_TILE,), jnp.int32),
            pltpu.VMEM((88, D // 2), jnp.int32),
            pltpu.VMEM((88, D // 2), jnp.int32),
            pltpu.VMEM((INIT_PER_TILE,), jnp.int32),
            pltpu.SemaphoreType.DMA,
            pltpu.SemaphoreType.DMA,
            pltpu.SemaphoreType.DMA,
            pltpu.SemaphoreType.DMA,
            pltpu.SemaphoreType.DMA,
            pltpu.SemaphoreType.DMA,
        ],
    )
    return f(dest_t, tok_t, x_bf)


def _ffn_body(buf_ref, w1_ref, b1_ref, w2_ref, b2_ref, out_ref):
    xb = buf_ref[...].astype(jnp.float32)
    h = (jnp.dot(xb, w1_ref[0], preferred_element_type=jnp.float32)
         + b1_ref[0, 0, :])
    h = jax.nn.gelu(h)
    out_ref[...] = (jnp.dot(h, w2_ref[0], preferred_element_type=jnp.float32)
                    + b2_ref[0, 0, :])


def _run_ffn(buf, w1, b1r, w2, b2r):
    emap = lambda i: jnp.minimum(i // 5, E - 1)
    return pl.pallas_call(
        _ffn_body,
        grid=(NB,),
        in_specs=[
            pl.BlockSpec((128, D), lambda i: (i, 0)),
            pl.BlockSpec((1, D, DFF), lambda i: (emap(i), 0, 0)),
            pl.BlockSpec((1, 1, DFF), lambda i: (emap(i), 0, 0)),
            pl.BlockSpec((1, DFF, D), lambda i: (emap(i), 0, 0)),
            pl.BlockSpec((1, 1, D), lambda i: (emap(i), 0, 0)),
        ],
        out_specs=pl.BlockSpec((128, D), lambda i: (i, 0)),
        out_shape=jax.ShapeDtypeStruct((S_PAD, D), jnp.float32),
        compiler_params=pltpu.CompilerParams(
            dimension_semantics=("arbitrary",)),
    )(buf, w1, b1r, w2, b2r)


_GDN = lax.GatherDimensionNumbers(offset_dims=(), collapsed_slice_dims=(0,),
                                  start_index_map=(0,))


def _splat(vec16, kk):
    return lax.gather(vec16, kk[:, None], _GDN, (1,),
                      mode=lax.GatherScatterMode.PROMISE_IN_BOUNDS)


def _combine_body(cidx_hbm, gate_hbm, scaled_hbm, out_hbm,
                  idx0_v, idx1_v, gv0_v, gv1_v, r0a, r1a, r0b, r1b, oa, ob,
                  sem_ga, sem_gb, sem_wa, sem_wb):
    cid = lax.axis_index("c")
    sid = lax.axis_index("s")
    wid = sid * NC + cid
    t0 = wid * 64
    pltpu.sync_copy(cidx_hbm.at[0, pl.ds(t0, 64)], idx0_v)
    pltpu.sync_copy(cidx_hbm.at[1, pl.ds(t0, 64)], idx1_v)
    pltpu.sync_copy(gate_hbm.at[0, pl.ds(t0, 64)], gv0_v)
    pltpu.sync_copy(gate_hbm.at[1, pl.ds(t0, 64)], gv1_v)
    r0s = (r0a, r0b)
    r1s = (r1a, r1b)
    outs = (oa, ob)
    gsems = (sem_ga, sem_gb)
    wsems = (sem_wa, sem_wb)
    zero16 = jnp.zeros((16,), jnp.int32)

    def start_gather(c):
        p = c % 2
        ga = pltpu.async_copy(
            scaled_hbm.at[idx0_v.at[pl.ds(c * 16, 16)]], r0s[p], gsems[p])
        gb = pltpu.async_copy(
            scaled_hbm.at[idx1_v.at[pl.ds(c * 16, 16)]], r1s[p], gsems[p])
        return ga, gb

    def add_and_write(c):
        p = c % 2
        g0c = gv0_v[pl.ds(c * 16, 16)]
        g1c = gv1_v[pl.ds(c * 16, 16)]

        def add_row(k, _):
            kk = zero16 + k
            ga = _splat(g0c, kk)
            gb = _splat(g1c, kk)
            for v in range(D // 16):
                sl = pl.ds(v * 16, 16)
                outs[p][k, sl] = r0s[p][k, sl] * ga + r1s[p][k, sl] * gb
            return 0

        lax.fori_loop(0, 16, add_row, 0)
        return pltpu.async_copy(
            outs[p], out_hbm.at[pl.ds(t0 + c * 16, 16)], wsems[p])

    pend_g = [None, None]
    pend_w = [None, None]
    pend_g[0] = start_gather(0)
    pend_g[1] = start_gather(1)
    for c in range(4):
        p = c % 2
        pend_g[p][0].wait()
        pend_g[p][1].wait()
        if pend_w[p] is not None:
            pend_w[p].wait()
        pend_w[p] = add_and_write(c)
        if c + 2 < 4:
            pend_g[p] = start_gather(c + 2)
    pend_w[0].wait()
    pend_w[1].wait()


def _run_combine(dest_t, gate_t, scaled):
    mesh = plsc.VectorSubcoreMesh(core_axis_name="c", subcore_axis_name="s")
    f = pl.kernel(
        _combine_body,
        out_type=jax.ShapeDtypeStruct((T, D), jnp.float32),
        mesh=mesh,
        scratch_types=[
            pltpu.VMEM((64,), jnp.int32),
            pltpu.VMEM((64,), jnp.int32),
            pltpu.VMEM((64,), jnp.float32),
            pltpu.VMEM((64,), jnp.float32),
            pltpu.VMEM((16, D), jnp.float32),
            pltpu.VMEM((16, D), jnp.float32),
            pltpu.VMEM((16, D), jnp.float32),
            pltpu.VMEM((16, D), jnp.float32),
            pltpu.VMEM((16, D), jnp.float32),
            pltpu.VMEM((16, D), jnp.float32),
            pltpu.SemaphoreType.DMA,
            pltpu.SemaphoreType.DMA,
            pltpu.SemaphoreType.DMA,
            pltpu.SemaphoreType.DMA,
        ],
    )
    return f(dest_t, gate_t, scaled)


@jax.jit
def kernel(x, Wg, W1, b1, W2, b2):
    dest_t, gate_t, x_bf = _run_router(x, Wg)
    tok_t = jnp.broadcast_to(jnp.arange(T, dtype=jnp.int32)[None, :], (2, T))
    x_pk = lax.bitcast_convert_type(x_bf.reshape(T, D // 2, 2), jnp.int32)
    buf_pk = _run_dispatch(dest_t, tok_t, x_pk)
    buf = lax.bitcast_convert_type(buf_pk, jnp.bfloat16).reshape(S_PAD, D)
    b1r = b1.reshape(E, 1, DFF)
    b2r = b2.reshape(E, 1, D)
    scaled = _run_ffn(buf, W1, b1r, W2, b2r)
    return _run_combine(dest_t, gate_t, scaled)


# speedup vs baseline: 1.6025x; 1.6025x over previous
import jax
import jax.numpy as jnp
from jax import lax
from jax.experimental import pallas as pl
from jax.experimental.pallas import tpu as pltpu
from jax.experimental.pallas import tpu_sc as plsc

T = 2048
D = 1024
DFF = 2048
E = 8
C = 640
S = E * C
S_PAD = 5376
N_TRASH = S_PAD - S
NB = S_PAD // 128

NC = 2
NS = 16
HALF = S_PAD // NC
PER_TILE = HALF // NS
INIT_PER_TILE = S_PAD // NS


def _router_body(x_ref, wg_ref, dest_ref, gate_ref):
    x = x_ref[...]
    logits = jnp.dot(x, wg_ref[...], preferred_element_type=jnp.float32)
    col = lax.broadcasted_iota(jnp.int32, (T, E), 1)
    m = jnp.max(logits, axis=1, keepdims=True)
    p = jnp.exp(logits - m)
    probs = p / jnp.sum(p, axis=1, keepdims=True)
    p0 = jnp.max(probs, axis=1, keepdims=True)
    i0 = jnp.min(jnp.where(probs == p0, col, E), axis=1, keepdims=True)
    mask0 = col == i0
    probs2 = jnp.where(mask0, -1.0, probs)
    p1 = jnp.max(probs2, axis=1, keepdims=True)
    i1 = jnp.min(jnp.where(probs2 == p1, col, E), axis=1, keepdims=True)
    mask1 = col == i1
    sm = p0 + p1 + 1e-9
    g0 = p0 / sm
    g1 = p1 / sm
    cnt = mask0.astype(jnp.bfloat16) + mask1.astype(jnp.bfloat16)
    ri = lax.broadcasted_iota(jnp.int32, (T, T), 0)
    cj = lax.broadcasted_iota(jnp.int32, (T, T), 1)
    tri = (cj < ri).astype(jnp.bfloat16)
    pref = jnp.dot(tri, cnt, preferred_element_type=jnp.float32)
    pos0 = jnp.sum(jnp.where(mask0, pref, 0.0), axis=1, keepdims=True)
    pos1 = jnp.sum(jnp.where(mask1, pref, 0.0), axis=1, keepdims=True)
    keep0 = pos0 < C
    keep1 = pos1 < C
    t2 = lax.broadcasted_iota(jnp.int32, (T, 1), 0) * 2
    i0f = i0.astype(jnp.float32)
    i1f = i1.astype(jnp.float32)
    d0 = jnp.where(keep0, i0f * C + pos0,
                   (S + (t2 % N_TRASH)).astype(jnp.float32))
    d1 = jnp.where(keep1, i1f * C + pos1,
                   (S + ((t2 + 1) % N_TRASH)).astype(jnp.float32))
    gv0 = jnp.where(keep0, g0, 0.0)
    gv1 = jnp.where(keep1, g1, 0.0)
    col8 = lax.broadcasted_iota(jnp.int32, (T, 8), 1)
    packed = jnp.where(col8 == 0, d0,
             jnp.where(col8 == 1, d1,
             jnp.where(col8 == 2, gv0,
             jnp.where(col8 == 3, gv1, 0.0))))
    pt = packed.T
    dest_ref[...] = pt[0:2, :].astype(jnp.int32)
    gate_ref[...] = pt[2:4, :]


def _run_router(x, Wg):
    return pl.pallas_call(
        _router_body,
        out_shape=[jax.ShapeDtypeStruct((2, T), jnp.int32),
                   jax.ShapeDtypeStruct((2, T), jnp.float32)],
    )(x, Wg)


def _dispatch_body(dest_hbm, tok_hbm, x_hbm, buf_hbm,
                   tfs_sp, idx_v, tok_v, tfs_v, rows_a, rows_b, binit_i,
                   sem_in, sem_sp, sem_ga, sem_gb, sem_wa, sem_wb):
    cid = lax.axis_index("c")
    sid = lax.axis_index("s")
    t0 = sid * 128
    c1 = pltpu.async_copy(dest_hbm.at[:, pl.ds(t0, 128)], idx_v, sem_in)
    c2 = pltpu.async_copy(tok_hbm.at[:, pl.ds(t0, 128)], tok_v, sem_in)
    zi = jnp.zeros((16,), jnp.int32)
    for k in range(INIT_PER_TILE // 16):
        binit_i[pl.ds(k * 16, 16)] = zi
    o = sid * INIT_PER_TILE
    i1 = pltpu.async_copy(binit_i, tfs_sp.at[pl.ds(o, INIT_PER_TILE)], sem_sp)
    c1.wait()
    c2.wait()
    i1.wait()
    plsc.subcore_barrier()
    s0 = pltpu.async_copy(tok_v.at[0], tfs_sp.at[idx_v.at[0]], sem_sp)
    s1 = pltpu.async_copy(tok_v.at[1], tfs_sp.at[idx_v.at[1]], sem_sp)
    s0.wait()
    s1.wait()
    plsc.subcore_barrier()
    g0 = cid * HALF + sid * PER_TILE
    pltpu.sync_copy(tfs_sp.at[pl.ds(g0, PER_TILE)], tfs_v)
    offs = (0, 48, 96, 144)
    szs = (48, 48, 48, 24)
    bufs = (rows_a, rows_b)
    gsems = (sem_ga, sem_gb)
    wsems = (sem_wa, sem_wb)

    def start_gather(c):
        p = c % 2
        return pltpu.async_copy(
            x_hbm.at[tfs_v.at[pl.ds(offs[c], szs[c])]],
            bufs[p].at[pl.ds(0, szs[c])], gsems[p])

    def start_write(c):
        p = c % 2
        return pltpu.async_copy(
            bufs[p].at[pl.ds(0, szs[c])],
            buf_hbm.at[pl.ds(g0 + offs[c], szs[c])], wsems[p])

    g_a = start_gather(0)
    g_b = start_gather(1)
    g_a.wait()
    w_a = start_write(0)
    g_b.wait()
    w_b = start_write(1)
    w_a.wait()
    g_a = start_gather(2)
    w_b.wait()
    g_b = start_gather(3)
    g_a.wait()
    w_a = start_write(2)
    g_b.wait()
    w_b = start_write(3)
    w_a.wait()
    w_b.wait()


def _run_dispatch(dest_t, tok_t, x):
    mesh = plsc.VectorSubcoreMesh(core_axis_name="c", subcore_axis_name="s")
    f = pl.kernel(
        _dispatch_body,
        out_type=jax.ShapeDtypeStruct((S_PAD, D), jnp.float32),
        mesh=mesh,
        scratch_types=[
            pltpu.VMEM_SHARED((S_PAD,), jnp.int32),
            pltpu.VMEM((2, 128), jnp.int32),
            pltpu.VMEM((2, 128), jnp.int32),
            pltpu.VMEM((PER_TILE,), jnp.int32),
            pltpu.VMEM((48, D), jnp.float32),
            pltpu.VMEM((48, D), jnp.float32),
            pltpu.VMEM((INIT_PER_TILE,), jnp.int32),
            pltpu.SemaphoreType.DMA,
            pltpu.SemaphoreType.DMA,
            pltpu.SemaphoreType.DMA,
            pltpu.SemaphoreType.DMA,
            pltpu.SemaphoreType.DMA,
            pltpu.SemaphoreType.DMA,
        ],
    )
    return f(dest_t, tok_t, x)


def _ffn_body(buf_ref, w1_ref, b1_ref, w2_ref, b2_ref, out_ref):
    e = jnp.minimum(pl.program_id(0) // 5, E - 1)
    xb = buf_ref[...]
    h = (jnp.dot(xb, w1_ref[0], preferred_element_type=jnp.float32)
         + b1_ref[pl.ds(e, 1), :])
    h = jax.nn.gelu(h)
    out_ref[...] = (jnp.dot(h, w2_ref[0], preferred_element_type=jnp.float32)
                    + b2_ref[pl.ds(e, 1), :])


def _run_ffn(buf, w1, b1, w2, b2):
    emap = lambda i: jnp.minimum(i // 5, E - 1)
    return pl.pallas_call(
        _ffn_body,
        grid=(NB,),
        in_specs=[
            pl.BlockSpec((128, D), lambda i: (i, 0)),
            pl.BlockSpec((1, D, DFF), lambda i: (emap(i), 0, 0)),
            pl.BlockSpec((E, DFF), lambda i: (0, 0)),
            pl.BlockSpec((1, DFF, D), lambda i: (emap(i), 0, 0)),
            pl.BlockSpec((E, D), lambda i: (0, 0)),
        ],
        out_specs=pl.BlockSpec((128, D), lambda i: (i, 0)),
        out_shape=jax.ShapeDtypeStruct((S_PAD, D), jnp.float32),
        compiler_params=pltpu.CompilerParams(
            dimension_semantics=("arbitrary",)),
    )(buf, w1, b1, w2, b2)


_GDN = lax.GatherDimensionNumbers(offset_dims=(), collapsed_slice_dims=(0,),
                                  start_index_map=(0,))


def _splat(vec16, kk):
    return lax.gather(vec16, kk[:, None], _GDN, (1,),
                      mode=lax.GatherScatterMode.PROMISE_IN_BOUNDS)


def _combine_body(cidx_hbm, gate_hbm, scaled_hbm, out_hbm,
                  idx0_v, idx1_v, gv0_v, gv1_v, r0a, r1a, r0b, r1b, oa, ob,
                  sem_ga, sem_gb, sem_wa, sem_wb):
    cid = lax.axis_index("c")
    sid = lax.axis_index("s")
    wid = sid * NC + cid
    t0 = wid * 64
    pltpu.sync_copy(cidx_hbm.at[0, pl.ds(t0, 64)], idx0_v)
    pltpu.sync_copy(cidx_hbm.at[1, pl.ds(t0, 64)], idx1_v)
    pltpu.sync_copy(gate_hbm.at[0, pl.ds(t0, 64)], gv0_v)
    pltpu.sync_copy(gate_hbm.at[1, pl.ds(t0, 64)], gv1_v)
    r0s = (r0a, r0b)
    r1s = (r1a, r1b)
    outs = (oa, ob)
    gsems = (sem_ga, sem_gb)
    wsems = (sem_wa, sem_wb)
    zero16 = jnp.zeros((16,), jnp.int32)

    def start_gather(c):
        p = c % 2
        ga = pltpu.async_copy(
            scaled_hbm.at[idx0_v.at[pl.ds(c * 16, 16)]], r0s[p], gsems[p])
        gb = pltpu.async_copy(
            scaled_hbm.at[idx1_v.at[pl.ds(c * 16, 16)]], r1s[p], gsems[p])
        return ga, gb

    def add_and_write(c):
        p = c % 2
        g0c = gv0_v[pl.ds(c * 16, 16)]
        g1c = gv1_v[pl.ds(c * 16, 16)]

        def add_row(k, _):
            kk = zero16 + k
            ga = _splat(g0c, kk)
            gb = _splat(g1c, kk)
            for v in range(D // 16):
                sl = pl.ds(v * 16, 16)
                outs[p][k, sl] = r0s[p][k, sl] * ga + r1s[p][k, sl] * gb
            return 0

        lax.fori_loop(0, 16, add_row, 0)
        return pltpu.async_copy(
            outs[p], out_hbm.at[pl.ds(t0 + c * 16, 16)], wsems[p])

    pend_g = [None, None]
    pend_w = [None, None]
    pend_g[0] = start_gather(0)
    pend_g[1] = start_gather(1)
    for c in range(4):
        p = c % 2
        pend_g[p][0].wait()
        pend_g[p][1].wait()
        if pend_w[p] is not None:
            pend_w[p].wait()
        pend_w[p] = add_and_write(c)
        if c + 2 < 4:
            pend_g[p] = start_gather(c + 2)
    pend_w[0].wait()
    pend_w[1].wait()


def _run_combine(dest_t, gate_t, scaled):
    mesh = plsc.VectorSubcoreMesh(core_axis_name="c", subcore_axis_name="s")
    f = pl.kernel(
        _combine_body,
        out_type=jax.ShapeDtypeStruct((T, D), jnp.float32),
        mesh=mesh,
        scratch_types=[
            pltpu.VMEM((64,), jnp.int32),
            pltpu.VMEM((64,), jnp.int32),
            pltpu.VMEM((64,), jnp.float32),
            pltpu.VMEM((64,), jnp.float32),
            pltpu.VMEM((16, D), jnp.float32),
            pltpu.VMEM((16, D), jnp.float32),
            pltpu.VMEM((16, D), jnp.float32),
            pltpu.VMEM((16, D), jnp.float32),
            pltpu.VMEM((16, D), jnp.float32),
            pltpu.VMEM((16, D), jnp.float32),
            pltpu.SemaphoreType.DMA,
            pltpu.SemaphoreType.DMA,
            pltpu.SemaphoreType.DMA,
            pltpu.SemaphoreType.DMA,
        ],
    )
    return f(dest_t, gate_t, scaled)


@jax.jit
def kernel(x, Wg, W1, b1, W2, b2):
    dest_t, gate_t = _run_router(x, Wg)
    tok_t = jnp.broadcast_to(jnp.arange(T, dtype=jnp.int32)[None, :], (2, T))
    buf = _run_dispatch(dest_t, tok_t, x)
    scaled = _run_ffn(buf, W1, b1, W2, b2)
    return _run_combine(dest_t, gate_t, scaled)


# speedup vs baseline: 2.2649x; 1.4133x over previous
import jax
import jax.numpy as jnp
from jax import lax
from jax.experimental import pallas as pl
from jax.experimental.pallas import tpu as pltpu
from jax.experimental.pallas import tpu_sc as plsc

T = 2048
D = 1024
DFF = 2048
E = 8
C = 640
S = E * C
S_PAD = 5376
N_TRASH = S_PAD - S
NB = S_PAD // 128

NC = 2
NS = 16
HALF = S_PAD // NC
PER_TILE = HALF // NS
INIT_PER_TILE = S_PAD // NS


def _router_body(x_ref, wg_ref, dest_ref, gate_ref):
    x = x_ref[...]
    logits = jnp.dot(x, wg_ref[...], preferred_element_type=jnp.float32)
    col = lax.broadcasted_iota(jnp.int32, (T, E), 1)
    m = jnp.max(logits, axis=1, keepdims=True)
    p = jnp.exp(logits - m)
    probs = p / jnp.sum(p, axis=1, keepdims=True)
    p0 = jnp.max(probs, axis=1, keepdims=True)
    i0 = jnp.min(jnp.where(probs == p0, col, E), axis=1, keepdims=True)
    mask0 = col == i0
    probs2 = jnp.where(mask0, -1.0, probs)
    p1 = jnp.max(probs2, axis=1, keepdims=True)
    i1 = jnp.min(jnp.where(probs2 == p1, col, E), axis=1, keepdims=True)
    mask1 = col == i1
    sm = p0 + p1 + 1e-9
    g0 = p0 / sm
    g1 = p1 / sm
    cnt = mask0.astype(jnp.bfloat16) + mask1.astype(jnp.bfloat16)
    ri = lax.broadcasted_iota(jnp.int32, (T, T), 0)
    cj = lax.broadcasted_iota(jnp.int32, (T, T), 1)
    tri = (cj < ri).astype(jnp.bfloat16)
    pref = jnp.dot(tri, cnt, preferred_element_type=jnp.float32)
    pos0 = jnp.sum(jnp.where(mask0, pref, 0.0), axis=1, keepdims=True)
    pos1 = jnp.sum(jnp.where(mask1, pref, 0.0), axis=1, keepdims=True)
    keep0 = pos0 < C
    keep1 = pos1 < C
    t2 = lax.broadcasted_iota(jnp.int32, (T, 1), 0) * 2
    i0f = i0.astype(jnp.float32)
    i1f = i1.astype(jnp.float32)
    d0 = jnp.where(keep0, i0f * C + pos0,
                   (S + (t2 % N_TRASH)).astype(jnp.float32))
    d1 = jnp.where(keep1, i1f * C + pos1,
                   (S + ((t2 + 1) % N_TRASH)).astype(jnp.float32))
    gv0 = jnp.where(keep0, g0, 0.0)
    gv1 = jnp.where(keep1, g1, 0.0)
    col8 = lax.broadcasted_iota(jnp.int32, (T, 8), 1)
    packed = jnp.where(col8 == 0, d0,
             jnp.where(col8 == 1, d1,
             jnp.where(col8 == 2, gv0,
             jnp.where(col8 == 3, gv1, 0.0))))
    pt = packed.T
    dest_ref[...] = pt[0:2, :].astype(jnp.int32)
    gate_ref[...] = pt[2:4, :]


def _run_router(x, Wg):
    return pl.pallas_call(
        _router_body,
        out_shape=[jax.ShapeDtypeStruct((2, T), jnp.int32),
                   jax.ShapeDtypeStruct((2, T), jnp.float32)],
    )(x, Wg)


def _dispatch_body(dest_hbm, x_hbm, buf_hbm,
                   tfs_sp, idx_v, tok_v, tfs_v, rows_a, rows_b, binit_i,
                   sem_in, sem_sp, sem_ga, sem_gb, sem_wa, sem_wb):
    cid = lax.axis_index("c")
    sid = lax.axis_index("s")
    t0 = sid * 128
    c1 = pltpu.async_copy(dest_hbm.at[:, pl.ds(t0, 128)], idx_v, sem_in)
    lane = lax.broadcasted_iota(jnp.int32, (16,), 0)
    o = sid * INIT_PER_TILE
    for k in range(INIT_PER_TILE // 16):
        binit_i[pl.ds(k * 16, 16)] = (lane + (o + k * 16)) & (T - 1)
    for j in range(2):
        for s in range(8):
            tok_v[j, pl.ds(s * 16, 16)] = lane + (t0 + s * 16)
    i1 = pltpu.async_copy(binit_i, tfs_sp.at[pl.ds(o, INIT_PER_TILE)], sem_sp)
    c1.wait()
    i1.wait()
    plsc.subcore_barrier()
    s0 = pltpu.async_copy(tok_v.at[0], tfs_sp.at[idx_v.at[0]], sem_sp)
    s1 = pltpu.async_copy(tok_v.at[1], tfs_sp.at[idx_v.at[1]], sem_sp)
    s0.wait()
    s1.wait()
    plsc.subcore_barrier()
    g0 = cid * HALF + sid * PER_TILE
    pltpu.sync_copy(tfs_sp.at[pl.ds(g0, PER_TILE)], tfs_v)
    offs = (0, 56, 112)
    szs = (56, 56, 56)
    bufs = (rows_a, rows_b)
    gsems = (sem_ga, sem_gb)
    wsems = (sem_wa, sem_wb)

    def start_gather(c):
        p = c % 2
        return pltpu.async_copy(
            x_hbm.at[tfs_v.at[pl.ds(offs[c], szs[c])]],
            bufs[p].at[pl.ds(0, szs[c])], gsems[p])

    def start_write(c):
        p = c % 2
        return pltpu.async_copy(
            bufs[p].at[pl.ds(0, szs[c])],
            buf_hbm.at[pl.ds(g0 + offs[c], szs[c])], wsems[p])

    g_a = start_gather(0)
    g_b = start_gather(1)
    g_a.wait()
    w_a = start_write(0)
    g_b.wait()
    w_b = start_write(1)
    w_a.wait()
    g_a = start_gather(2)
    g_a.wait()
    w_a = start_write(2)
    w_b.wait()
    w_a.wait()


def _run_dispatch(dest_t, x):
    mesh = plsc.VectorSubcoreMesh(core_axis_name="c", subcore_axis_name="s")
    f = pl.kernel(
        _dispatch_body,
        out_type=jax.ShapeDtypeStruct((S_PAD, D), jnp.float32),
        mesh=mesh,
        scratch_types=[
            pltpu.VMEM_SHARED((S_PAD,), jnp.int32),
            pltpu.VMEM((2, 128), jnp.int32),
            pltpu.VMEM((2, 128), jnp.int32),
            pltpu.VMEM((PER_TILE,), jnp.int32),
            pltpu.VMEM((56, D), jnp.float32),
            pltpu.VMEM((56, D), jnp.float32),
            pltpu.VMEM((INIT_PER_TILE,), jnp.int32),
            pltpu.SemaphoreType.DMA,
            pltpu.SemaphoreType.DMA,
            pltpu.SemaphoreType.DMA,
            pltpu.SemaphoreType.DMA,
            pltpu.SemaphoreType.DMA,
            pltpu.SemaphoreType.DMA,
        ],
    )
    return f(dest_t, x)


def _ffn_body(buf_ref, w1_ref, b1_ref, w2_ref, b2_ref, out_ref):
    e = jnp.minimum(pl.program_id(0) // 5, E - 1)
    xb = buf_ref[...]
    h = (jnp.dot(xb, w1_ref[0], preferred_element_type=jnp.float32)
         + b1_ref[pl.ds(e, 1), :])
    h = jax.nn.gelu(h)
    out_ref[...] = (jnp.dot(h, w2_ref[0], preferred_element_type=jnp.float32)
                    + b2_ref[pl.ds(e, 1), :])


def _run_ffn(buf, w1, b1, w2, b2):
    emap = lambda i: jnp.minimum(i // 5, E - 1)
    return pl.pallas_call(
        _ffn_body,
        grid=(NB,),
        in_specs=[
            pl.BlockSpec((128, D), lambda i: (i, 0)),
            pl.BlockSpec((1, D, DFF), lambda i: (emap(i), 0, 0)),
            pl.BlockSpec((E, DFF), lambda i: (0, 0)),
            pl.BlockSpec((1, DFF, D), lambda i: (emap(i), 0, 0)),
            pl.BlockSpec((E, D), lambda i: (0, 0)),
        ],
        out_specs=pl.BlockSpec((128, D), lambda i: (i, 0)),
        out_shape=jax.ShapeDtypeStruct((S_PAD, D), jnp.float32),
        compiler_params=pltpu.CompilerParams(
            dimension_semantics=("arbitrary",)),
    )(buf, w1, b1, w2, b2)


_GDN = lax.GatherDimensionNumbers(offset_dims=(), collapsed_slice_dims=(0,),
                                  start_index_map=(0,))


def _splat(vec16, kk):
    return lax.gather(vec16, kk[:, None], _GDN, (1,),
                      mode=lax.GatherScatterMode.PROMISE_IN_BOUNDS)


def _combine_body(cidx_hbm, gate_hbm, scaled_hbm, out_hbm,
                  idx0_v, idx1_v, gv0_v, gv1_v, r0a, r1a, r0b, r1b, oa, ob,
                  sem_ga, sem_gb, sem_wa, sem_wb):
    cid = lax.axis_index("c")
    sid = lax.axis_index("s")
    wid = sid * NC + cid
    t0 = wid * 64
    pltpu.sync_copy(cidx_hbm.at[0, pl.ds(t0, 64)], idx0_v)
    pltpu.sync_copy(cidx_hbm.at[1, pl.ds(t0, 64)], idx1_v)
    pltpu.sync_copy(gate_hbm.at[0, pl.ds(t0, 64)], gv0_v)
    pltpu.sync_copy(gate_hbm.at[1, pl.ds(t0, 64)], gv1_v)
    r0s = (r0a, r0b)
    r1s = (r1a, r1b)
    outs = (oa, ob)
    gsems = (sem_ga, sem_gb)
    wsems = (sem_wa, sem_wb)
    zero16 = jnp.zeros((16,), jnp.int32)

    def start_gather(c):
        p = c % 2
        ga = pltpu.async_copy(
            scaled_hbm.at[idx0_v.at[pl.ds(c * 16, 16)]], r0s[p], gsems[p])
        gb = pltpu.async_copy(
            scaled_hbm.at[idx1_v.at[pl.ds(c * 16, 16)]], r1s[p], gsems[p])
        return ga, gb

    def add_and_write(c):
        p = c % 2
        g0c = gv0_v[pl.ds(c * 16, 16)]
        g1c = gv1_v[pl.ds(c * 16, 16)]

        def add_row(k, _):
            kk = zero16 + k
            ga = _splat(g0c, kk)
            gb = _splat(g1c, kk)
            for v in range(D // 16):
                sl = pl.ds(v * 16, 16)
                outs[p][k, sl] = r0s[p][k, sl] * ga + r1s[p][k, sl] * gb
            return 0

        lax.fori_loop(0, 16, add_row, 0)
        return pltpu.async_copy(
            outs[p], out_hbm.at[pl.ds(t0 + c * 16, 16)], wsems[p])

    pend_g = [None, None]
    pend_w = [None, None]
    pend_g[0] = start_gather(0)
    pend_g[1] = start_gather(1)
    for c in range(4):
        p = c % 2
        pend_g[p][0].wait()
        pend_g[p][1].wait()
        if pend_w[p] is not None:
            pend_w[p].wait()
        pend_w[p] = add_and_write(c)
        if c + 2 < 4:
            pend_g[p] = start_gather(c + 2)
    pend_w[0].wait()
    pend_w[1].wait()


def _run_combine(dest_t, gate_t, scaled):
    mesh = plsc.VectorSubcoreMesh(core_axis_name="c", subcore_axis_name="s")
    f = pl.kernel(
        _combine_body,
        out_type=jax.ShapeDtypeStruct((T, D), jnp.float32),
        mesh=mesh,
        scratch_types=[
            pltpu.VMEM((64,), jnp.int32),
            pltpu.VMEM((64,), jnp.int32),
            pltpu.VMEM((64,), jnp.float32),
            pltpu.VMEM((64,), jnp.float32),
            pltpu.VMEM((16, D), jnp.float32),
            pltpu.VMEM((16, D), jnp.float32),
            pltpu.VMEM((16, D), jnp.float32),
            pltpu.VMEM((16, D), jnp.float32),
            pltpu.VMEM((16, D), jnp.float32),
            pltpu.VMEM((16, D), jnp.float32),
            pltpu.SemaphoreType.DMA,
            pltpu.SemaphoreType.DMA,
            pltpu.SemaphoreType.DMA,
            pltpu.SemaphoreType.DMA,
        ],
    )
    return f(dest_t, gate_t, scaled)


@jax.jit
def kernel(x, Wg, W1, b1, W2, b2):
    dest_t, gate_t = _run_router(x, Wg)
    buf = _run_dispatch(dest_t, x)
    scaled = _run_ffn(buf, W1, b1, W2, b2)
    return _run_combine(dest_t, gate_t, scaled)


# speedup vs baseline: 3.0788x; 1.3593x over previous
import jax
import jax.numpy as jnp
from jax import lax
from jax.experimental import pallas as pl
from jax.experimental.pallas import tpu as pltpu
from jax.experimental.pallas import tpu_sc as plsc

T = 2048
D = 1024
DFF = 2048
E = 8
C = 640
S = E * C
S_PAD = 5376
N_TRASH = S_PAD - S
NB = S_PAD // 128

NC = 2
NS = 16
HALF = S_PAD // NC
PER_TILE = HALF // NS
INIT_PER_TILE = S_PAD // NS


def _router_body(x_ref, wg_ref, dest_ref, gate_ref):
    x = x_ref[...]
    logits = jnp.dot(x, wg_ref[...], preferred_element_type=jnp.float32)
    col = lax.broadcasted_iota(jnp.int32, (T, E), 1)
    m = jnp.max(logits, axis=1, keepdims=True)
    p = jnp.exp(logits - m)
    probs = p / jnp.sum(p, axis=1, keepdims=True)
    p0 = jnp.max(probs, axis=1, keepdims=True)
    i0 = jnp.min(jnp.where(probs == p0, col, E), axis=1, keepdims=True)
    mask0 = col == i0
    probs2 = jnp.where(mask0, -1.0, probs)
    p1 = jnp.max(probs2, axis=1, keepdims=True)
    i1 = jnp.min(jnp.where(probs2 == p1, col, E), axis=1, keepdims=True)
    mask1 = col == i1
    sm = p0 + p1 + 1e-9
    g0 = p0 / sm
    g1 = p1 / sm
    cnt = mask0.astype(jnp.bfloat16) + mask1.astype(jnp.bfloat16)
    ri = lax.broadcasted_iota(jnp.int32, (T, T), 0)
    cj = lax.broadcasted_iota(jnp.int32, (T, T), 1)
    tri = (cj < ri).astype(jnp.bfloat16)
    pref = jnp.dot(tri, cnt, preferred_element_type=jnp.float32)
    pos0 = jnp.sum(jnp.where(mask0, pref, 0.0), axis=1, keepdims=True)
    pos1 = jnp.sum(jnp.where(mask1, pref, 0.0), axis=1, keepdims=True)
    keep0 = pos0 < C
    keep1 = pos1 < C
    t2 = lax.broadcasted_iota(jnp.int32, (T, 1), 0) * 2
    i0f = i0.astype(jnp.float32)
    i1f = i1.astype(jnp.float32)
    d0 = jnp.where(keep0, i0f * C + pos0,
                   (S + (t2 % N_TRASH)).astype(jnp.float32))
    d1 = jnp.where(keep1, i1f * C + pos1,
                   (S + ((t2 + 1) % N_TRASH)).astype(jnp.float32))
    gv0 = jnp.where(keep0, g0, 0.0)
    gv1 = jnp.where(keep1, g1, 0.0)
    col8 = lax.broadcasted_iota(jnp.int32, (T, 8), 1)
    packed = jnp.where(col8 == 0, d0,
             jnp.where(col8 == 1, d1,
             jnp.where(col8 == 2, gv0,
             jnp.where(col8 == 3, gv1, 0.0))))
    pt = packed.T
    dest_ref[...] = pt[0:2, :].astype(jnp.int32)
    gate_ref[...] = pt[2:4, :]


def _run_router(x, Wg):
    return pl.pallas_call(
        _router_body,
        out_shape=[jax.ShapeDtypeStruct((2, T), jnp.int32),
                   jax.ShapeDtypeStruct((2, T), jnp.float32)],
    )(x, Wg)


def _dispatch_body(dest_hbm, x_hbm, buf_hbm,
                   tfs_sp, idx_v, tok_v, tfs_v, rows_a, rows_b, binit_i,
                   sem_in, sem_sp, sem_ga, sem_gb, sem_wa, sem_wb):
    cid = lax.axis_index("c")
    sid = lax.axis_index("s")
    t0 = sid * 128
    c1 = pltpu.async_copy(dest_hbm.at[:, pl.ds(t0, 128)], idx_v, sem_in)
    lane = lax.broadcasted_iota(jnp.int32, (16,), 0)
    o = sid * INIT_PER_TILE
    for k in range(INIT_PER_TILE // 16):
        binit_i[pl.ds(k * 16, 16)] = (lane + (o + k * 16)) & (T - 1)
    for j in range(2):
        for s in range(8):
            tok_v[j, pl.ds(s * 16, 16)] = lane + (t0 + s * 16)
    i1 = pltpu.async_copy(binit_i, tfs_sp.at[pl.ds(o, INIT_PER_TILE)], sem_sp)
    c1.wait()
    i1.wait()
    plsc.subcore_barrier()
    s0 = pltpu.async_copy(tok_v.at[0], tfs_sp.at[idx_v.at[0]], sem_sp)
    s1 = pltpu.async_copy(tok_v.at[1], tfs_sp.at[idx_v.at[1]], sem_sp)
    s0.wait()
    s1.wait()
    plsc.subcore_barrier()
    g0 = cid * HALF + sid * PER_TILE
    pltpu.sync_copy(tfs_sp.at[pl.ds(g0, PER_TILE)], tfs_v)
    offs = (0, 56, 112)
    szs = (56, 56, 56)
    bufs = (rows_a, rows_b)
    gsems = (sem_ga, sem_gb)
    wsems = (sem_wa, sem_wb)

    def start_gather(c):
        p = c % 2
        return pltpu.async_copy(
            x_hbm.at[tfs_v.at[pl.ds(offs[c], szs[c])]],
            bufs[p].at[pl.ds(0, szs[c])], gsems[p])

    def start_write(c):
        p = c % 2
        return pltpu.async_copy(
            bufs[p].at[pl.ds(0, szs[c])],
            buf_hbm.at[pl.ds(g0 + offs[c], szs[c])], wsems[p])

    g_a = start_gather(0)
    g_b = start_gather(1)
    g_a.wait()
    w_a = start_write(0)
    g_b.wait()
    w_b = start_write(1)
    w_a.wait()
    g_a = start_gather(2)
    g_a.wait()
    w_a = start_write(2)
    w_b.wait()
    w_a.wait()


def _run_dispatch(dest_t, x):
    mesh = plsc.VectorSubcoreMesh(core_axis_name="c", subcore_axis_name="s")
    f = pl.kernel(
        _dispatch_body,
        out_type=jax.ShapeDtypeStruct((S_PAD, D), jnp.float32),
        mesh=mesh,
        scratch_types=[
            pltpu.VMEM_SHARED((S_PAD,), jnp.int32),
            pltpu.VMEM((2, 128), jnp.int32),
            pltpu.VMEM((2, 128), jnp.int32),
            pltpu.VMEM((PER_TILE,), jnp.int32),
            pltpu.VMEM((56, D), jnp.float32),
            pltpu.VMEM((56, D), jnp.float32),
            pltpu.VMEM((INIT_PER_TILE,), jnp.int32),
            pltpu.SemaphoreType.DMA,
            pltpu.SemaphoreType.DMA,
            pltpu.SemaphoreType.DMA,
            pltpu.SemaphoreType.DMA,
            pltpu.SemaphoreType.DMA,
            pltpu.SemaphoreType.DMA,
        ],
    )
    return f(dest_t, x)


def _ffn_body(buf_ref, w1_ref, b1_ref, w2_ref, b2_ref, out_ref):
    e = pl.program_id(0)
    xb = buf_ref[...]
    h = (jnp.dot(xb, w1_ref[0], preferred_element_type=jnp.float32)
         + b1_ref[pl.ds(e, 1), :])
    h = jax.nn.gelu(h)
    out_ref[...] = (jnp.dot(h, w2_ref[0], preferred_element_type=jnp.float32)
                    + b2_ref[pl.ds(e, 1), :])


def _run_ffn(buf, w1, b1, w2, b2):
    return pl.pallas_call(
        _ffn_body,
        grid=(E,),
        in_specs=[
            pl.BlockSpec((C, D), lambda i: (i, 0)),
            pl.BlockSpec((1, D, DFF), lambda i: (i, 0, 0)),
            pl.BlockSpec((E, DFF), lambda i: (0, 0)),
            pl.BlockSpec((1, DFF, D), lambda i: (i, 0, 0)),
            pl.BlockSpec((E, D), lambda i: (0, 0)),
        ],
        out_specs=pl.BlockSpec((C, D), lambda i: (i, 0)),
        out_shape=jax.ShapeDtypeStruct((S, D), jnp.float32),
        compiler_params=pltpu.CompilerParams(
            dimension_semantics=("arbitrary",)),
    )(buf, w1, b1, w2, b2)


_GDN = lax.GatherDimensionNumbers(offset_dims=(), collapsed_slice_dims=(0,),
                                  start_index_map=(0,))


def _splat(vec16, kk):
    return lax.gather(vec16, kk[:, None], _GDN, (1,),
                      mode=lax.GatherScatterMode.PROMISE_IN_BOUNDS)


def _combine_body(cidx_hbm, gate_hbm, scaled_hbm, out_hbm,
                  idx0_v, idx1_v, gv0_v, gv1_v, r0a, r1a, r0b, r1b, oa, ob,
                  sem_ga, sem_gb, sem_wa, sem_wb):
    cid = lax.axis_index("c")
    sid = lax.axis_index("s")
    wid = sid * NC + cid
    t0 = wid * 64
    pltpu.sync_copy(cidx_hbm.at[0, pl.ds(t0, 64)], idx0_v)
    pltpu.sync_copy(cidx_hbm.at[1, pl.ds(t0, 64)], idx1_v)
    pltpu.sync_copy(gate_hbm.at[0, pl.ds(t0, 64)], gv0_v)
    pltpu.sync_copy(gate_hbm.at[1, pl.ds(t0, 64)], gv1_v)
    smax = jnp.zeros((16,), jnp.int32) + (S - 1)
    for s in range(4):
        sl = pl.ds(s * 16, 16)
        idx0_v[sl] = jnp.minimum(idx0_v[sl], smax)
        idx1_v[sl] = jnp.minimum(idx1_v[sl], smax)
    r0s = (r0a, r0b)
    r1s = (r1a, r1b)
    outs = (oa, ob)
    gsems = (sem_ga, sem_gb)
    wsems = (sem_wa, sem_wb)
    zero16 = jnp.zeros((16,), jnp.int32)

    def start_gather(c):
        p = c % 2
        ga = pltpu.async_copy(
            scaled_hbm.at[idx0_v.at[pl.ds(c * 16, 16)]], r0s[p], gsems[p])
        gb = pltpu.async_copy(
            scaled_hbm.at[idx1_v.at[pl.ds(c * 16, 16)]], r1s[p], gsems[p])
        return ga, gb

    def add_and_write(c):
        p = c % 2
        g0c = gv0_v[pl.ds(c * 16, 16)]
        g1c = gv1_v[pl.ds(c * 16, 16)]

        def add_row(k, _):
            kk = zero16 + k
            ga = _splat(g0c, kk)
            gb = _splat(g1c, kk)
            for v in range(D // 16):
                sl = pl.ds(v * 16, 16)
                outs[p][k, sl] = r0s[p][k, sl] * ga + r1s[p][k, sl] * gb
            return 0

        lax.fori_loop(0, 16, add_row, 0)
        return pltpu.async_copy(
            outs[p], out_hbm.at[pl.ds(t0 + c * 16, 16)], wsems[p])

    pend_g = [None, None]
    pend_w = [None, None]
    pend_g[0] = start_gather(0)
    pend_g[1] = start_gather(1)
    for c in range(4):
        p = c % 2
        pend_g[p][0].wait()
        pend_g[p][1].wait()
        if pend_w[p] is not None:
            pend_w[p].wait()
        pend_w[p] = add_and_write(c)
        if c + 2 < 4:
            pend_g[p] = start_gather(c + 2)
    pend_w[0].wait()
    pend_w[1].wait()


def _run_combine(dest_t, gate_t, scaled):
    mesh = plsc.VectorSubcoreMesh(core_axis_name="c", subcore_axis_name="s")
    f = pl.kernel(
        _combine_body,
        out_type=jax.ShapeDtypeStruct((T, D), jnp.float32),
        mesh=mesh,
        scratch_types=[
            pltpu.VMEM((64,), jnp.int32),
            pltpu.VMEM((64,), jnp.int32),
            pltpu.VMEM((64,), jnp.float32),
            pltpu.VMEM((64,), jnp.float32),
            pltpu.VMEM((16, D), jnp.float32),
            pltpu.VMEM((16, D), jnp.float32),
            pltpu.VMEM((16, D), jnp.float32),
            pltpu.VMEM((16, D), jnp.float32),
            pltpu.VMEM((16, D), jnp.float32),
            pltpu.VMEM((16, D), jnp.float32),
            pltpu.SemaphoreType.DMA,
            pltpu.SemaphoreType.DMA,
            pltpu.SemaphoreType.DMA,
            pltpu.SemaphoreType.DMA,
        ],
    )
    return f(dest_t, gate_t, scaled)


@jax.jit
def kernel(x, Wg, W1, b1, W2, b2):
    dest_t, gate_t = _run_router(x, Wg)
    buf = _run_dispatch(dest_t, x)
    scaled = _run_ffn(buf, W1, b1, W2, b2)
    return _run_combine(dest_t, gate_t, scaled)


# speedup vs baseline: 3.0929x; 1.0046x over previous
import jax
import jax.numpy as jnp
from jax import lax
from jax.experimental import pallas as pl
from jax.experimental.pallas import tpu as pltpu
from jax.experimental.pallas import tpu_sc as plsc

T = 2048
D = 1024
DFF = 2048
E = 8
C = 640
S = E * C
S_PAD = 5376
N_TRASH = S_PAD - S
NB = S_PAD // 128

NC = 2
NS = 16
HALF = S_PAD // NC
PER_TILE = HALF // NS
INIT_PER_TILE = S_PAD // NS


def _router_body(x_ref, wg_ref, dest_ref, gate_ref):
    x = x_ref[...]
    logits = jnp.dot(x, wg_ref[...], preferred_element_type=jnp.float32)
    col = lax.broadcasted_iota(jnp.int32, (T, E), 1)
    m = jnp.max(logits, axis=1, keepdims=True)
    p = jnp.exp(logits - m)
    probs = p / jnp.sum(p, axis=1, keepdims=True)
    p0 = jnp.max(probs, axis=1, keepdims=True)
    i0 = jnp.min(jnp.where(probs == p0, col, E), axis=1, keepdims=True)
    mask0 = col == i0
    probs2 = jnp.where(mask0, -1.0, probs)
    p1 = jnp.max(probs2, axis=1, keepdims=True)
    i1 = jnp.min(jnp.where(probs2 == p1, col, E), axis=1, keepdims=True)
    mask1 = col == i1
    sm = p0 + p1 + 1e-9
    g0 = p0 / sm
    g1 = p1 / sm
    cnt = mask0.astype(jnp.bfloat16) + mask1.astype(jnp.bfloat16)
    ri = lax.broadcasted_iota(jnp.int32, (T, T), 0)
    cj = lax.broadcasted_iota(jnp.int32, (T, T), 1)
    tri = (cj < ri).astype(jnp.bfloat16)
    pref = jnp.dot(tri, cnt, preferred_element_type=jnp.float32)
    pos0 = jnp.sum(jnp.where(mask0, pref, 0.0), axis=1, keepdims=True)
    pos1 = jnp.sum(jnp.where(mask1, pref, 0.0), axis=1, keepdims=True)
    keep0 = pos0 < C
    keep1 = pos1 < C
    t2 = lax.broadcasted_iota(jnp.int32, (T, 1), 0) * 2
    i0f = i0.astype(jnp.float32)
    i1f = i1.astype(jnp.float32)
    d0 = jnp.where(keep0, i0f * C + pos0,
                   (S + (t2 % N_TRASH)).astype(jnp.float32))
    d1 = jnp.where(keep1, i1f * C + pos1,
                   (S + ((t2 + 1) % N_TRASH)).astype(jnp.float32))
    gv0 = jnp.where(keep0, g0, 0.0)
    gv1 = jnp.where(keep1, g1, 0.0)
    col8 = lax.broadcasted_iota(jnp.int32, (T, 8), 1)
    packed = jnp.where(col8 == 0, d0,
             jnp.where(col8 == 1, d1,
             jnp.where(col8 == 2, gv0,
             jnp.where(col8 == 3, gv1, 0.0))))
    pt = packed.T
    dest_ref[...] = pt[0:2, :].astype(jnp.int32)
    gate_ref[...] = pt[2:4, :]


def _run_router(x, Wg):
    return pl.pallas_call(
        _router_body,
        out_shape=[jax.ShapeDtypeStruct((2, T), jnp.int32),
                   jax.ShapeDtypeStruct((2, T), jnp.float32)],
    )(x, Wg)


def _dispatch_body(dest_hbm, x_hbm, buf_hbm,
                   tfs_sp, idx_v, tok_v, tfs_v, rows_a, rows_b, binit_i,
                   sem_in, sem_sp, sem_ga, sem_gb, sem_wa, sem_wb):
    cid = lax.axis_index("c")
    sid = lax.axis_index("s")
    t0 = sid * 128
    c1 = pltpu.async_copy(dest_hbm.at[:, pl.ds(t0, 128)], idx_v, sem_in)
    lane = lax.broadcasted_iota(jnp.int32, (16,), 0)
    o = sid * INIT_PER_TILE
    for k in range(INIT_PER_TILE // 16):
        binit_i[pl.ds(k * 16, 16)] = (lane + (o + k * 16)) & (T - 1)
    for j in range(2):
        for s in range(8):
            tok_v[j, pl.ds(s * 16, 16)] = lane + (t0 + s * 16)
    i1 = pltpu.async_copy(binit_i, tfs_sp.at[pl.ds(o, INIT_PER_TILE)], sem_sp)
    c1.wait()
    i1.wait()
    plsc.subcore_barrier()
    s0 = pltpu.async_copy(tok_v.at[0], tfs_sp.at[idx_v.at[0]], sem_sp)
    s1 = pltpu.async_copy(tok_v.at[1], tfs_sp.at[idx_v.at[1]], sem_sp)
    s0.wait()
    s1.wait()
    plsc.subcore_barrier()
    g0 = cid * (S // NC) + sid * (S // NC // NS)
    pltpu.sync_copy(tfs_sp.at[pl.ds(g0, S // NC // NS)], tfs_v)
    offs = (0, 56, 112)
    szs = (56, 56, 48)
    bufs = (rows_a, rows_b)
    gsems = (sem_ga, sem_gb)
    wsems = (sem_wa, sem_wb)

    def start_gather(c):
        p = c % 2
        return pltpu.async_copy(
            x_hbm.at[tfs_v.at[pl.ds(offs[c], szs[c])]],
            bufs[p].at[pl.ds(0, szs[c])], gsems[p])

    def start_write(c):
        p = c % 2
        return pltpu.async_copy(
            bufs[p].at[pl.ds(0, szs[c])],
            buf_hbm.at[pl.ds(g0 + offs[c], szs[c])], wsems[p])

    g_a = start_gather(0)
    g_b = start_gather(1)
    g_a.wait()
    w_a = start_write(0)
    g_b.wait()
    w_b = start_write(1)
    w_a.wait()
    g_a = start_gather(2)
    g_a.wait()
    w_a = start_write(2)
    w_b.wait()
    w_a.wait()


def _run_dispatch(dest_t, x):
    mesh = plsc.VectorSubcoreMesh(core_axis_name="c", subcore_axis_name="s")
    f = pl.kernel(
        _dispatch_body,
        out_type=jax.ShapeDtypeStruct((S, D), jnp.float32),
        mesh=mesh,
        scratch_types=[
            pltpu.VMEM_SHARED((S_PAD,), jnp.int32),
            pltpu.VMEM((2, 128), jnp.int32),
            pltpu.VMEM((2, 128), jnp.int32),
            pltpu.VMEM((S // NC // NS,), jnp.int32),
            pltpu.VMEM((56, D), jnp.float32),
            pltpu.VMEM((56, D), jnp.float32),
            pltpu.VMEM((INIT_PER_TILE,), jnp.int32),
            pltpu.SemaphoreType.DMA,
            pltpu.SemaphoreType.DMA,
            pltpu.SemaphoreType.DMA,
            pltpu.SemaphoreType.DMA,
            pltpu.SemaphoreType.DMA,
            pltpu.SemaphoreType.DMA,
        ],
    )
    return f(dest_t, x)


def _ffn_body(buf_ref, w1_ref, b1_ref, w2_ref, b2_ref, out_ref):
    e = pl.program_id(0)
    xb = buf_ref[...]
    h = (jnp.dot(xb, w1_ref[0], preferred_element_type=jnp.float32)
         + b1_ref[pl.ds(e, 1), :])
    h = jax.nn.gelu(h)
    out_ref[...] = (jnp.dot(h, w2_ref[0], preferred_element_type=jnp.float32)
                    + b2_ref[pl.ds(e, 1), :])


def _run_ffn(buf, w1, b1, w2, b2):
    return pl.pallas_call(
        _ffn_body,
        grid=(E,),
        in_specs=[
            pl.BlockSpec((C, D), lambda i: (i, 0)),
            pl.BlockSpec((1, D, DFF), lambda i: (i, 0, 0)),
            pl.BlockSpec((E, DFF), lambda i: (0, 0)),
            pl.BlockSpec((1, DFF, D), lambda i: (i, 0, 0)),
            pl.BlockSpec((E, D), lambda i: (0, 0)),
        ],
        out_specs=pl.BlockSpec((C, D), lambda i: (i, 0)),
        out_shape=jax.ShapeDtypeStruct((S, D), jnp.float32),
        compiler_params=pltpu.CompilerParams(
            dimension_semantics=("arbitrary",)),
    )(buf, w1, b1, w2, b2)


_GDN = lax.GatherDimensionNumbers(offset_dims=(), collapsed_slice_dims=(0,),
                                  start_index_map=(0,))


def _splat(vec16, kk):
    return lax.gather(vec16, kk[:, None], _GDN, (1,),
                      mode=lax.GatherScatterMode.PROMISE_IN_BOUNDS)


def _combine_body(cidx_hbm, gate_hbm, scaled_hbm, out_hbm,
                  idx0_v, idx1_v, gv0_v, gv1_v, r0a, r1a, r0b, r1b, oa, ob,
                  sem_ga, sem_gb, sem_wa, sem_wb):
    cid = lax.axis_index("c")
    sid = lax.axis_index("s")
    wid = sid * NC + cid
    t0 = wid * 64
    pltpu.sync_copy(cidx_hbm.at[0, pl.ds(t0, 64)], idx0_v)
    pltpu.sync_copy(cidx_hbm.at[1, pl.ds(t0, 64)], idx1_v)
    pltpu.sync_copy(gate_hbm.at[0, pl.ds(t0, 64)], gv0_v)
    pltpu.sync_copy(gate_hbm.at[1, pl.ds(t0, 64)], gv1_v)
    smax = jnp.zeros((16,), jnp.int32) + (S - 1)
    for s in range(4):
        sl = pl.ds(s * 16, 16)
        idx0_v[sl] = jnp.minimum(idx0_v[sl], smax)
        idx1_v[sl] = jnp.minimum(idx1_v[sl], smax)
    r0s = (r0a, r0b)
    r1s = (r1a, r1b)
    outs = (oa, ob)
    gsems = (sem_ga, sem_gb)
    wsems = (sem_wa, sem_wb)
    zero16 = jnp.zeros((16,), jnp.int32)

    def start_gather(c):
        p = c % 2
        ga = pltpu.async_copy(
            scaled_hbm.at[idx0_v.at[pl.ds(c * 16, 16)]], r0s[p], gsems[p])
        gb = pltpu.async_copy(
            scaled_hbm.at[idx1_v.at[pl.ds(c * 16, 16)]], r1s[p], gsems[p])
        return ga, gb

    def add_and_write(c):
        p = c % 2
        g0c = gv0_v[pl.ds(c * 16, 16)]
        g1c = gv1_v[pl.ds(c * 16, 16)]

        def add_row(k, _):
            kk = zero16 + k
            ga = _splat(g0c, kk)
            gb = _splat(g1c, kk)
            for v in range(D // 16):
                sl = pl.ds(v * 16, 16)
                outs[p][k, sl] = r0s[p][k, sl] * ga + r1s[p][k, sl] * gb
            return 0

        lax.fori_loop(0, 16, add_row, 0)
        return pltpu.async_copy(
            outs[p], out_hbm.at[pl.ds(t0 + c * 16, 16)], wsems[p])

    pend_g = [None, None]
    pend_w = [None, None]
    pend_g[0] = start_gather(0)
    pend_g[1] = start_gather(1)
    for c in range(4):
        p = c % 2
        pend_g[p][0].wait()
        pend_g[p][1].wait()
        if pend_w[p] is not None:
            pend_w[p].wait()
        pend_w[p] = add_and_write(c)
        if c + 2 < 4:
            pend_g[p] = start_gather(c + 2)
    pend_w[0].wait()
    pend_w[1].wait()


def _run_combine(dest_t, gate_t, scaled):
    mesh = plsc.VectorSubcoreMesh(core_axis_name="c", subcore_axis_name="s")
    f = pl.kernel(
        _combine_body,
        out_type=jax.ShapeDtypeStruct((T, D), jnp.float32),
        mesh=mesh,
        scratch_types=[
            pltpu.VMEM((64,), jnp.int32),
            pltpu.VMEM((64,), jnp.int32),
            pltpu.VMEM((64,), jnp.float32),
            pltpu.VMEM((64,), jnp.float32),
            pltpu.VMEM((16, D), jnp.float32),
            pltpu.VMEM((16, D), jnp.float32),
            pltpu.VMEM((16, D), jnp.float32),
            pltpu.VMEM((16, D), jnp.float32),
            pltpu.VMEM((16, D), jnp.float32),
            pltpu.VMEM((16, D), jnp.float32),
            pltpu.SemaphoreType.DMA,
            pltpu.SemaphoreType.DMA,
            pltpu.SemaphoreType.DMA,
            pltpu.SemaphoreType.DMA,
        ],
    )
    return f(dest_t, gate_t, scaled)


@jax.jit
def kernel(x, Wg, W1, b1, W2, b2):
    dest_t, gate_t = _run_router(x, Wg)
    buf = _run_dispatch(dest_t, x)
    scaled = _run_ffn(buf, W1, b1, W2, b2)
    return _run_combine(dest_t, gate_t, scaled)
